# baseline (speedup 1.0000x reference)
import jax
import jax.numpy as jnp
from jax import lax
from jax.experimental import pallas as pl
from jax.experimental.pallas import tpu as pltpu
from jax.experimental.pallas import tpu_sc as plsc

_B, _N, _C = 2, 64, 128
_D = _H = _W = 32
_NROI = _B * _N
_OD, _OH, _OW = 4, 8, 8
_NBINS = _OD * _OH * _OW
_BPU = 2
_RPB = 16
_RPU = _BPU * _RPB
_NUNITS = _NBINS // _BPU
_VOX = _D * _H * _W
_PW = 4 * (_C // 2)
_RING = 4


def _floor_f(x):
    t = x.astype(jnp.int32).astype(jnp.float32)
    return jnp.where(x < t, t - 1.0, t)


def _corner_vecs(p, c1, bin_sz, dim_f):
    t = p >> 1
    a = p & 1
    tf = t.astype(jnp.float32)
    af = a.astype(jnp.float32)
    g = c1 + (tf + 0.5) * 0.5 * bin_sz
    gn = (g + 0.5) * (1.0 / dim_f) * 2.0 - 1.0
    i = ((gn + 1.0) * dim_f - 1.0) * 0.5
    fl = _floor_f(i)
    cf = fl + af
    w1 = i - fl
    w = jnp.where(a == 1, w1, 1.0 - w1)
    valid = (cf >= 0.0) & (cf <= dim_f - 1.0)
    w = jnp.where(valid, w, 0.0)
    ci = jnp.clip(cf, 0.0, dim_f - 1.0).astype(jnp.int32)
    return ci, w


def _axis_patch(t, c1, bin_sz, dim_f):
    tf = t.astype(jnp.float32)
    g = c1 + (tf + 0.5) * 0.5 * bin_sz
    gn = (g + 0.5) * (1.0 / dim_f) * 2.0 - 1.0
    i = ((gn + 1.0) * dim_f - 1.0) * 0.5
    fl = _floor_f(i)
    w1 = i - fl
    w0 = 1.0 - w1
    w0 = jnp.where((fl >= 0.0) & (fl <= dim_f - 1.0), w0, 0.0)
    w1 = jnp.where((fl + 1.0 >= 0.0) & (fl + 1.0 <= dim_f - 1.0), w1, 0.0)
    base_f = jnp.clip(fl, 0.0, dim_f - 2.0)
    ws0 = jnp.where(fl == base_f, w0,
                    jnp.where(fl + 1.0 == base_f, w1, 0.0))
    ws1 = jnp.where(fl == base_f, w1,
                    jnp.where(fl == base_f + 1.0, w0, 0.0))
    return base_f.astype(jnp.int32), ws0, ws1


def _roialign_body(table, roispad, out, roi_v, zi, zw, yb, yw0, yw1,
                   xb, xw0, xw1, outst, *ring):
    idxb = ring[0:_RING]
    wb = ring[_RING:2 * _RING]
    rows = ring[2 * _RING:3 * _RING]
    sems = ring[3 * _RING:4 * _RING]
    info = plsc.get_sparse_core_info()
    nc = info.num_cores
    wid = lax.axis_index("s") * nc + lax.axis_index("c")
    io16 = lax.broadcasted_iota(jnp.int32, (16,), 0)
    io512 = io16 * (2 * _NBINS)

    def roi_body(q, carry):
        rid = wid * 4 + q
        base_row = (rid >> 6) * _VOX
        pltpu.sync_copy(roispad.at[rid], roi_v)
        rv = roi_v[...] * 32.0 - 0.5
        z1, y1, x1 = rv[0], rv[1], rv[2]
        z2, y2, x2 = rv[3], rv[4], rv[5]
        bin_d = jnp.maximum(z2 - z1, 1e-6) * 0.25
        bin_h = jnp.maximum(y2 - y1, 1e-6) * 0.125
        bin_w = jnp.maximum(x2 - x1, 1e-6) * 0.125
        ziv, zwv = _corner_vecs(io16, z1, bin_d, 32.0)
        zi[...] = ziv
        zw[...] = zwv
        b, s0, s1 = _axis_patch(io16, y1, bin_h, 32.0)
        yb[...] = b
        yw0[...] = s0
        yw1[...] = s1
        b, s0, s1 = _axis_patch(io16, x1, bin_w, 32.0)
        xb[...] = b
        xw0[...] = s0
        xw1[...] = s1

        def build_unit(u, s):
            for q2 in range(_BPU):
                binid = u * _BPU + q2
                ii = binid >> 6
                jj = (binid >> 3) & 7
                kk = binid & 7
                zpos = ii * 4 + (io16 >> 2)
                ypos = jj * 2 + ((io16 >> 1) & 1)
                xpos = kk * 2 + (io16 & 1)
                zz = plsc.load_gather(zi, [zpos])
                zwv_ = plsc.load_gather(zw, [zpos])
                ybg = plsc.load_gather(yb, [ypos])
                y0g = plsc.load_gather(yw0, [ypos])
                y1g = plsc.load_gather(yw1, [ypos])
                xbg = plsc.load_gather(xb, [xpos])
                x0g = plsc.load_gather(xw0, [xpos])
                x1g = plsc.load_gather(xw1, [xpos])
                idxb[s][pl.ds(q2 * 16, 16)] = (
                    zz * 1024 + ybg * 32 + xbg + base_row)
                zs = zwv_ * 0.125
                wb[s][pl.ds(0 * 32 + q2 * 16, 16)] = zs * y0g * x0g
                wb[s][pl.ds(1 * 32 + q2 * 16, 16)] = zs * y0g * x1g
                wb[s][pl.ds(2 * 32 + q2 * 16, 16)] = zs * y1g * x0g
                wb[s][pl.ds(3 * 32 + q2 * 16, 16)] = zs * y1g * x1g

        def fire_unit(s):
            pltpu.async_copy(table.at[idxb[s]], rows[s], sems[s])

        def drain_unit(u, s):
            pltpu.make_async_copy(table.at[idxb[s]], rows[s],
                                  sems[s]).wait()
            for q2 in range(_BPU):
                binid = u * _BPU + q2

                def acc_body(r, acc, _q2=q2, _s=s):
                    rr = _q2 * 16 + r
                    ws = [plsc.load_gather(
                        wb[_s], [jnp.full((16,), c * 32 + rr, jnp.int32)])
                        for c in range(4)]
                    new = list(acc)
                    for part in range(4):
                        for g in range(4):
                            xi = rows[_s][rr, pl.ds(part * 64 + g * 16, 16)]
                            lo = plsc.bitcast(xi << 16, jnp.float32)
                            hi = plsc.bitcast(xi & jnp.int32(-65536),
                                              jnp.float32)
                            new[2 * g] = new[2 * g] + ws[part] * lo
                            new[2 * g + 1] = new[2 * g + 1] + ws[part] * hi
                    return tuple(new)

                acc = lax.fori_loop(
                    0, _RPB, acc_body,
                    tuple(jnp.zeros((16,), jnp.float32) for _ in range(8)))
                for g in range(4):
                    for h in range(2):
                        plsc.store_scatter(
                            outst,
                            [io512 + ((32 * g + h) * _NBINS + binid)],
                            acc[2 * g + h])

        def ring_body(it, rcarry):
            for s in range(_RING):
                @pl.when(it > 0)
                def _drain(_s=s, _it=it):
                    drain_unit((_it - 1) * _RING + _s, _s)
            for s in range(_RING):
                @pl.when(it < _NUNITS // _RING)
                def _build(_s=s, _it=it):
                    build_unit(_it * _RING + _s, _s)
                    fire_unit(_s)
            return rcarry

        lax.fori_loop(0, _NUNITS // _RING + 1, ring_body, 0)
        pltpu.sync_copy(outst, out.at[rid])
        return carry

    lax.fori_loop(0, _NROI // 32, roi_body, 0)


def kernel(features, rois):
    f = features.astype(jnp.bfloat16).transpose(0, 2, 3, 4, 1)
    fx = jnp.concatenate([f[:, :, :, 1:], f[:, :, :, -1:]], axis=3)
    fy = jnp.concatenate([f[:, :, 1:], f[:, :, -1:]], axis=2)
    fyx = jnp.concatenate([fy[:, :, :, 1:], fy[:, :, :, -1:]], axis=3)
    patch = jnp.concatenate([f, fx, fy, fyx], axis=-1)
    table = jax.lax.bitcast_convert_type(
        patch.reshape(_B * _VOX, _PW, 2), jnp.int32)
    roispad = jnp.pad(rois.reshape(_NROI, 6), ((0, 0), (0, 10)))
    mesh = plsc.VectorSubcoreMesh(core_axis_name="c", subcore_axis_name="s")
    run = pl.kernel(
        _roialign_body,
        out_type=jax.ShapeDtypeStruct((_NROI, _C * _NBINS), jnp.float32),
        mesh=mesh,
        compiler_params=pltpu.CompilerParams(needs_layout_passes=False,
                                             use_tc_tiling_on_sc=False),
        scratch_types=[
            pltpu.VMEM((16,), jnp.float32),
            pltpu.VMEM((16,), jnp.int32),
            pltpu.VMEM((16,), jnp.float32),
            pltpu.VMEM((16,), jnp.int32),
            pltpu.VMEM((16,), jnp.float32),
            pltpu.VMEM((16,), jnp.float32),
            pltpu.VMEM((16,), jnp.int32),
            pltpu.VMEM((16,), jnp.float32),
            pltpu.VMEM((16,), jnp.float32),
            pltpu.VMEM((_C * _NBINS,), jnp.float32),
        ] + [pltpu.VMEM((_RPU,), jnp.int32)] * _RING
          + [pltpu.VMEM((4 * _RPU,), jnp.float32)] * _RING
          + [pltpu.VMEM((_RPU, _PW), jnp.int32)] * _RING
          + [pltpu.SemaphoreType.DMA] * _RING,
    )
    out = run(table, roispad)
    return out.reshape(_B, _N, _C, _OD, _OH, _OW)


# speedup vs baseline: 1.9656x; 1.9656x over previous
import jax
import jax.numpy as jnp
from jax import lax
from jax.experimental import pallas as pl
from jax.experimental.pallas import tpu as pltpu
from jax.experimental.pallas import tpu_sc as plsc

_B, _N, _C = 2, 64, 128
_D = _H = _W = 32
_NROI = _B * _N
_OD, _OH, _OW = 4, 8, 8
_NBINS = _OD * _OH * _OW
_RPB = 64
_BPU = 2
_RPU = _BPU * _RPB
_NUNITS = _NBINS // _BPU
_VOX = _D * _H * _W


def _corner_vecs(p, c1, bin_sz, dim_f):
    t = p >> 1
    a = p & 1
    tf = t.astype(jnp.float32)
    af = a.astype(jnp.float32)
    step = (tf + 0.5) * 0.5
    g = c1 + step * bin_sz
    gn = (g + 0.5) * (1.0 / dim_f) * 2.0 - 1.0
    i = ((gn + 1.0) * dim_f - 1.0) * 0.5
    ti = i.astype(jnp.int32).astype(jnp.float32)
    fl = jnp.where(i < ti, ti - 1.0, ti)
    cf = fl + af
    w1 = i - fl
    w = jnp.where(a == 1, w1, 1.0 - w1)
    valid = (cf >= 0.0) & (cf <= dim_f - 1.0)
    w = jnp.where(valid, w, 0.0)
    ci = jnp.clip(cf, 0.0, dim_f - 1.0).astype(jnp.int32)
    return ci, w


_RING = 8


def _roialign_body(table, roispad, out, roi_v, zi, zw, yi, yw, xi, xw,
                   outst, *ring):
    idxb = ring[0:_RING]
    wb = ring[_RING:2 * _RING]
    rows = ring[2 * _RING:3 * _RING]
    sems = ring[3 * _RING:4 * _RING]
    info = plsc.get_sparse_core_info()
    nc = info.num_cores
    wid = lax.axis_index("s") * nc + lax.axis_index("c")
    io16 = lax.broadcasted_iota(jnp.int32, (16,), 0)
    io512 = io16 * (2 * _NBINS)

    def roi_body(q, carry):
        rid = wid * 4 + q
        base_row = (rid >> 6) * _VOX
        pltpu.sync_copy(roispad.at[rid], roi_v)
        rv = roi_v[...] * 32.0 - 0.5
        z1, y1, x1 = rv[0], rv[1], rv[2]
        z2, y2, x2 = rv[3], rv[4], rv[5]
        bin_d = jnp.maximum(z2 - z1, 1e-6) * 0.25
        bin_h = jnp.maximum(y2 - y1, 1e-6) * 0.125
        bin_w = jnp.maximum(x2 - x1, 1e-6) * 0.125
        ziv, zwv = _corner_vecs(io16, z1, bin_d, 32.0)
        zi[...] = ziv
        zw[...] = zwv
        for h in range(2):
            yiv, ywv = _corner_vecs(io16 + 16 * h, y1, bin_h, 32.0)
            yi[pl.ds(16 * h, 16)] = yiv
            yw[pl.ds(16 * h, 16)] = ywv
            xiv, xwv = _corner_vecs(io16 + 16 * h, x1, bin_w, 32.0)
            xi[pl.ds(16 * h, 16)] = xiv
            xw[pl.ds(16 * h, 16)] = xwv

        def build_unit(u, s):
            for q2 in range(_BPU):
                binid = u * _BPU + q2
                ii = binid >> 6
                jj = (binid >> 3) & 7
                kk = binid & 7
                ypos = jj * 4 + (io16 >> 2)
                xpos = kk * 4 + (io16 & 3)
                ygi = plsc.load_gather(yi, [ypos])
                ygw = plsc.load_gather(yw, [ypos])
                xgi = plsc.load_gather(xi, [xpos])
                xgw = plsc.load_gather(xw, [xpos])
                yxi = ygi * 32 + xgi + base_row
                yxw = ygw * xgw * 0.125
                for r in range(4):
                    zpos = jnp.full((16,), ii * 4 + r, jnp.int32)
                    zz = plsc.load_gather(zi, [zpos])
                    zwt = plsc.load_gather(zw, [zpos])
                    off = (q2 * 4 + r) * 16
                    idxb[s][pl.ds(off, 16)] = yxi + zz * 1024
                    wb[s][pl.ds(off, 16)] = yxw * zwt

        def fire_unit(s):
            pltpu.async_copy(table.at[idxb[s]], rows[s], sems[s])

        def drain_unit(u, s):
            pltpu.make_async_copy(table.at[idxb[s]], rows[s],
                                  sems[s]).wait()
            for q2 in range(_BPU):
                binid = u * _BPU + q2

                def acc_body(r, acc, _q2=q2, _s=s):
                    rr = _q2 * 64 + r
                    w = plsc.load_gather(wb[_s],
                                         [jnp.full((16,), rr, jnp.int32)])
                    new = []
                    for g in range(4):
                        xi = rows[_s][rr, pl.ds(g * 16, 16)]
                        lo = plsc.bitcast(xi << 16, jnp.float32)
                        hi = plsc.bitcast(xi & jnp.int32(-65536), jnp.float32)
                        new.append(acc[2 * g] + w * lo)
                        new.append(acc[2 * g + 1] + w * hi)
                    return tuple(new)

                acc = lax.fori_loop(
                    0, 64, acc_body,
                    tuple(jnp.zeros((16,), jnp.float32) for _ in range(8)))
                for g in range(4):
                    for h in range(2):
                        plsc.store_scatter(
                            outst,
                            [io512 + ((32 * g + h) * _NBINS + binid)],
                            acc[2 * g + h])

        def ring_body(it, rcarry):
            for s in range(_RING):
                @pl.when(it > 0)
                def _drain(_s=s, _it=it):
                    drain_unit((_it - 1) * _RING + _s, _s)

                @pl.when(it < _NUNITS // _RING)
                def _build(_s=s, _it=it):
                    build_unit(_it * _RING + _s, _s)
                    fire_unit(_s)
            return rcarry

        lax.fori_loop(0, _NUNITS // _RING + 1, ring_body, 0)
        pltpu.sync_copy(outst, out.at[rid])
        return carry

    lax.fori_loop(0, _NROI // 32, roi_body, 0)


def kernel(features, rois):
    table = jax.lax.bitcast_convert_type(
        features.astype(jnp.bfloat16)
        .transpose(0, 2, 3, 4, 1).reshape(_B * _VOX, _C // 2, 2),
        jnp.int32)
    roispad = jnp.pad(rois.reshape(_NROI, 6), ((0, 0), (0, 10)))
    mesh = plsc.VectorSubcoreMesh(core_axis_name="c", subcore_axis_name="s")
    run = pl.kernel(
        _roialign_body,
        out_type=jax.ShapeDtypeStruct((_NROI, _C * _NBINS), jnp.float32),
        mesh=mesh,
        compiler_params=pltpu.CompilerParams(needs_layout_passes=False,
                                             use_tc_tiling_on_sc=False),
        scratch_types=[
            pltpu.VMEM((16,), jnp.float32),
            pltpu.VMEM((16,), jnp.int32),
            pltpu.VMEM((16,), jnp.float32),
            pltpu.VMEM((32,), jnp.int32),
            pltpu.VMEM((32,), jnp.float32),
            pltpu.VMEM((32,), jnp.int32),
            pltpu.VMEM((32,), jnp.float32),
            pltpu.VMEM((_C * _NBINS,), jnp.float32),
        ] + [pltpu.VMEM((_RPU,), jnp.int32)] * _RING
          + [pltpu.VMEM((_RPU,), jnp.float32)] * _RING
          + [pltpu.VMEM((_RPU, _C // 2), jnp.int32)] * _RING
          + [pltpu.SemaphoreType.DMA] * _RING,
    )
    out = run(table, roispad)
    return out.reshape(_B, _N, _C, _OD, _OH, _OW)
